# Initial kernel scaffold; baseline (speedup 1.0000x reference)
import jax
import jax.numpy as jnp
from jax.experimental import pallas as pl


def kernel(x, edge_index, batch, ln0_g, ln0_b, W1, b1, ln1_g, ln1_b, W2, b2, ln2_g, ln2_b, Wc, bc):
    raise NotImplementedError("write your pallas kernel here")


# speedup vs baseline: 11.5281x; 11.5281x over previous
import functools

import jax
import jax.numpy as jnp
from jax import lax
from jax.experimental import pallas as pl
from jax.experimental.pallas import tpu as pltpu
from jax.experimental.pallas import tpu_sc as plsc

N = 10000
E = 320000
FEAT = 128
HID = 128
NCLASS = 16
NGRAPH = 64

NC, NS = 2, 16
NW = NC * NS
CHUNK = 128
CH = 79
EPT = CH * CHUNK
EPAD = NW * EPT
NACC = NS * 640
ZROWS = 80
DEGW = NS * 640
ROWS_OUT = 624
TAIL = N - NS * ROWS_OUT

def _deg_body(dst_hbm, out_hbm, dstbuf, ones_v, zeros_v, acc):
    c = lax.axis_index("c")
    s = lax.axis_index("s")
    wid = c * NS + s

    def fill_ones(i, carry):
        ones_v[pl.ds(i * 16, 16)] = jnp.full((16,), 1.0, jnp.float32)
        return carry

    lax.fori_loop(0, CHUNK // 16, fill_ones, 0)

    def fill_zeros(i, carry):
        zeros_v[pl.ds(i * 16, 16)] = jnp.zeros((16,), jnp.float32)
        return carry

    lax.fori_loop(0, 640 // 16, fill_zeros, 0)
    pltpu.sync_copy(zeros_v, acc.at[pl.ds(s * 640, 640)])
    pltpu.sync_copy(dst_hbm.at[wid], dstbuf)
    plsc.subcore_barrier()

    def chunk_body(j, carry):
        pltpu.sync_copy(ones_v, acc.at[dstbuf.at[j]], add=True)
        return carry

    lax.fori_loop(0, CH, chunk_body, 0)
    plsc.subcore_barrier()
    pltpu.sync_copy(acc.at[pl.ds(s * 640, 640)],
                    out_hbm.at[c, pl.ds(s * 640, 640)])


def _edge_body(src_hbm, dst_hbm, tab_hbm, out_hbm,
               srcbuf, dstbuf, rowbuf, zbuf, acc, sem):
    c = lax.axis_index("c")
    s = lax.axis_index("s")
    wid = c * NS + s

    def zero_row(r, carry):
        for k in range(FEAT // 16):
            zbuf[r, pl.ds(k * 16, 16)] = jnp.zeros((16,), jnp.float32)
        return carry

    lax.fori_loop(0, ZROWS, zero_row, 0)

    def zero_slice(k, carry):
        pltpu.sync_copy(zbuf, acc.at[pl.ds(s * 640 + k * ZROWS, ZROWS)])
        return carry

    lax.fori_loop(0, 640 // ZROWS, zero_slice, 0)
    pltpu.sync_copy(src_hbm.at[wid], srcbuf)
    pltpu.sync_copy(dst_hbm.at[wid], dstbuf)
    plsc.subcore_barrier()

    def chunk_body(j, carry):
        pltpu.async_copy(tab_hbm.at[srcbuf.at[j]], rowbuf, sem).wait()
        pltpu.sync_copy(rowbuf, acc.at[dstbuf.at[j]], add=True)
        return carry

    lax.fori_loop(0, CH, chunk_body, 0)
    plsc.subcore_barrier()
    pltpu.sync_copy(acc.at[pl.ds(s * ROWS_OUT, ROWS_OUT)],
                    out_hbm.at[c, pl.ds(s * ROWS_OUT, ROWS_OUT)])

    @pl.when(s == NS - 1)
    def _tail():
        pltpu.sync_copy(acc.at[pl.ds(NS * ROWS_OUT, TAIL)],
                        out_hbm.at[c, pl.ds(NS * ROWS_OUT, TAIL)])


@functools.cache
def _sc_kernels():
    mesh = plsc.VectorSubcoreMesh(
        core_axis_name="c", subcore_axis_name="s",
        num_cores=NC, num_subcores=NS)
    deg = pl.kernel(
        _deg_body,
        out_type=jax.ShapeDtypeStruct((NC, DEGW), jnp.float32),
        mesh=mesh,
        scratch_types=[
            pltpu.VMEM((CH, CHUNK), jnp.int32),
            pltpu.VMEM((CHUNK,), jnp.float32),
            pltpu.VMEM((640,), jnp.float32),
            pltpu.VMEM_SHARED((DEGW,), jnp.float32),
        ],
    )
    edge = pl.kernel(
        _edge_body,
        out_type=jax.ShapeDtypeStruct((NC, N, FEAT), jnp.float32),
        mesh=mesh,
        scratch_types=[
            pltpu.VMEM((CH, CHUNK), jnp.int32),
            pltpu.VMEM((CH, CHUNK), jnp.int32),
            pltpu.VMEM((CHUNK, FEAT), jnp.float32),
            pltpu.VMEM((ZROWS, FEAT), jnp.float32),
            pltpu.VMEM_SHARED((NACC, FEAT), jnp.float32),
            pltpu.SemaphoreType.DMA,
        ],
    )
    return deg, edge


def _prep_body(degp_ref, dinv_ref):
    deg = degp_ref[0:1, :] + degp_ref[1:2, :] + 1.0
    dinv_ref[...] = lax.rsqrt(deg)


_prep = pl.pallas_call(
    _prep_body,
    out_shape=jax.ShapeDtypeStruct((1, DEGW), jnp.float32),
)

R = 1000
GRID = N // R


def _tc1_body(x_ref, dinv_ref, g0_ref, b0_ref, w1t_ref, b1_ref, out_ref):
    h = x_ref[...]
    m = jnp.mean(h, axis=1, keepdims=True)
    hc = h - m
    v = jnp.mean(hc * hc, axis=1, keepdims=True)
    hn = hc * lax.rsqrt(v + 1e-5) * g0_ref[...] + b0_ref[...]
    hw = jnp.dot(hn, w1t_ref[...], preferred_element_type=jnp.float32)
    out_ref[...] = (hw + b1_ref[...]) * dinv_ref[...]


_tc1 = pl.pallas_call(
    _tc1_body,
    grid=(GRID,),
    in_specs=[
        pl.BlockSpec((R, FEAT), lambda i: (i, 0)),
        pl.BlockSpec((R, 1), lambda i: (i, 0)),
        pl.BlockSpec((1, FEAT), lambda i: (0, 0)),
        pl.BlockSpec((1, FEAT), lambda i: (0, 0)),
        pl.BlockSpec((FEAT, HID), lambda i: (0, 0)),
        pl.BlockSpec((1, HID), lambda i: (0, 0)),
    ],
    out_specs=pl.BlockSpec((R, HID), lambda i: (i, 0)),
    out_shape=jax.ShapeDtypeStruct((N, HID), jnp.float32),
)


def _tc2_body(p_ref, g_ref, dinv_ref, lng_ref, lnb_ref, w2t_ref, b2_ref,
              out_ref):
    dinv = dinv_ref[...]
    h = (p_ref[0] + p_ref[1] + g_ref[...]) * dinv
    m = jnp.mean(h, axis=1, keepdims=True)
    hc = h - m
    v = jnp.mean(hc * hc, axis=1, keepdims=True)
    hn = hc * lax.rsqrt(v + 1e-5) * lng_ref[...] + lnb_ref[...]
    hr = jnp.maximum(hn, 0.0)
    hw = jnp.dot(hr, w2t_ref[...], preferred_element_type=jnp.float32)
    out_ref[...] = (hw + b2_ref[...]) * dinv


_tc2 = pl.pallas_call(
    _tc2_body,
    grid=(GRID,),
    in_specs=[
        pl.BlockSpec((NC, R, HID), lambda i: (0, i, 0)),
        pl.BlockSpec((R, HID), lambda i: (i, 0)),
        pl.BlockSpec((R, 1), lambda i: (i, 0)),
        pl.BlockSpec((1, HID), lambda i: (0, 0)),
        pl.BlockSpec((1, HID), lambda i: (0, 0)),
        pl.BlockSpec((HID, HID), lambda i: (0, 0)),
        pl.BlockSpec((1, HID), lambda i: (0, 0)),
    ],
    out_specs=pl.BlockSpec((R, HID), lambda i: (i, 0)),
    out_shape=jax.ShapeDtypeStruct((N, HID), jnp.float32),
)


def _tc3_body(p_ref, g_ref, dinv_ref, lng_ref, lnb_ref, batch_ref, wct_ref,
              bc_ref, out_ref, acc_ref):
    i = pl.program_id(0)

    @pl.when(i == 0)
    def _init():
        acc_ref[...] = jnp.zeros_like(acc_ref)

    h = (p_ref[0] + p_ref[1] + g_ref[...]) * dinv_ref[...]
    m = jnp.mean(h, axis=1, keepdims=True)
    hc = h - m
    v = jnp.mean(hc * hc, axis=1, keepdims=True)
    hn = hc * lax.rsqrt(v + 1e-5) * lng_ref[...] + lnb_ref[...]
    hr = jnp.maximum(hn, 0.0)
    onehot = (batch_ref[...] ==
              lax.broadcasted_iota(jnp.int32, (R, NGRAPH), 1))
    acc_ref[...] += lax.dot_general(
        onehot.astype(jnp.float32), hr, (((0,), (0,)), ((), ())),
        preferred_element_type=jnp.float32)

    @pl.when(i == GRID - 1)
    def _finish():
        logits = jnp.dot(acc_ref[...], wct_ref[...],
                         preferred_element_type=jnp.float32) + bc_ref[...]
        mx = jnp.max(logits, axis=1, keepdims=True)
        ex = jnp.exp(logits - mx)
        lse = jnp.log(jnp.sum(ex, axis=1, keepdims=True))
        out_ref[...] = logits - mx - lse


_tc3 = pl.pallas_call(
    _tc3_body,
    grid=(GRID,),
    in_specs=[
        pl.BlockSpec((NC, R, HID), lambda i: (0, i, 0)),
        pl.BlockSpec((R, HID), lambda i: (i, 0)),
        pl.BlockSpec((R, 1), lambda i: (i, 0)),
        pl.BlockSpec((1, HID), lambda i: (0, 0)),
        pl.BlockSpec((1, HID), lambda i: (0, 0)),
        pl.BlockSpec((R, 1), lambda i: (i, 0)),
        pl.BlockSpec((HID, NCLASS), lambda i: (0, 0)),
        pl.BlockSpec((1, NCLASS), lambda i: (0, 0)),
    ],
    out_specs=pl.BlockSpec((NGRAPH, NCLASS), lambda i: (0, 0)),
    out_shape=jax.ShapeDtypeStruct((NGRAPH, NCLASS), jnp.float32),
    scratch_shapes=[pltpu.VMEM((NGRAPH, HID), jnp.float32)],
)


def kernel(x, edge_index, batch, ln0_g, ln0_b, W1, b1, ln1_g, ln1_b,
           W2, b2, ln2_g, ln2_b, Wc, bc):
    pad = EPAD - E
    srcp = jnp.concatenate(
        [edge_index[0], jnp.zeros((pad,), jnp.int32)]).reshape(NW, CH, CHUNK)
    dstp = jnp.concatenate(
        [edge_index[1], jnp.full((pad,), N, jnp.int32)]).reshape(NW, CH, CHUNK)

    deg_kernel, edge_kernel = _sc_kernels()
    degp = deg_kernel(dstp)
    dinv = _prep(degp).reshape(DEGW)[:N].reshape(N, 1)

    g1 = _tc1(x, dinv, ln0_g.reshape(1, FEAT), ln0_b.reshape(1, FEAT),
              W1.T, b1.reshape(1, HID))
    p1 = edge_kernel(srcp, dstp, g1)
    g2 = _tc2(p1, g1, dinv, ln1_g.reshape(1, HID), ln1_b.reshape(1, HID),
              W2.T, b2.reshape(1, HID))
    p2 = edge_kernel(srcp, dstp, g2)
    return _tc3(p2, g2, dinv, ln2_g.reshape(1, HID), ln2_b.reshape(1, HID),
                batch.reshape(N, 1), Wc.T, bc.reshape(1, NCLASS))
